# Initial kernel scaffold; baseline (speedup 1.0000x reference)
import jax
import jax.numpy as jnp
from jax.experimental import pallas as pl


def kernel(h, edge_index, edge_weight, W0, b0, W1, b1, W2, b2):
    raise NotImplementedError("write your pallas kernel here")


# speedup vs baseline: 3.2081x; 3.2081x over previous
import functools

import jax
import jax.numpy as jnp
from jax import lax
from jax.experimental import pallas as pl
from jax.experimental.pallas import tpu as pltpu
from jax.experimental.pallas import tpu_sc as plsc

_N = 10000
_E = 320000
_D = 128

_NC = 2
_NS = 16
_NW = _NC * _NS
_EPW = _E // _NW
_CHUNK = 80
_NCHUNK = _EPW // _CHUNK
_NP = 10240
_RPS = _NP // _NS
_ZC = 128
_NZ = _RPS // _ZC


def _spmm_body(h_hbm, src_hbm, dst_hbm, w_hbm, out_hbm,
               srcc_v, dstc_v, wc_v, rows_v, stage_v, acc_sh, sem):
    cid = lax.axis_index("c")
    sid = lax.axis_index("s")
    wid = sid * _NC + cid

    if True:
        def zrow(i, _):
            z = jnp.zeros((16,), jnp.float32)
            for j in range(_D // 16):
                stage_v[i, pl.ds(j * 16, 16)] = z
            return 0
        lax.fori_loop(0, _ZC, zrow, 0)
        for k in range(_NZ):
            pltpu.sync_copy(
                stage_v, acc_sh.at[pl.ds(sid * _RPS + k * _ZC, _ZC)])
        plsc.subcore_barrier()

        def chunk(g, _):
            pltpu.sync_copy(src_hbm.at[wid, g], srcc_v)
            pltpu.sync_copy(dst_hbm.at[wid, g], dstc_v)
            pltpu.sync_copy(w_hbm.at[wid, g], wc_v)
            pltpu.async_copy(h_hbm.at[srcc_v], rows_v, sem).wait()

            def row(i, _):
                wv = plsc.load_gather(wc_v, [jnp.full((16,), i, jnp.int32)])
                for j in range(_D // 16):
                    sl = pl.ds(j * 16, 16)
                    rows_v[i, sl] = rows_v[i, sl] * wv
                return 0
            lax.fori_loop(0, _CHUNK, row, 0)

            pltpu.sync_copy(rows_v, acc_sh.at[dstc_v], add=True)
            return 0
        lax.fori_loop(0, _NCHUNK, chunk, 0)
        plsc.subcore_barrier()

        for k in range(_NZ):
            sl = pl.ds(sid * _RPS + k * _ZC, _ZC)
            pltpu.sync_copy(acc_sh.at[sl], out_hbm.at[cid, sl])


@functools.cache
def _sc_spmm():
    return pl.kernel(
        _spmm_body,
        out_type=jax.ShapeDtypeStruct((_NC, _NP, _D), jnp.float32),
        mesh=plsc.VectorSubcoreMesh(core_axis_name="c",
                                    subcore_axis_name="s"),
        compiler_params=pltpu.CompilerParams(needs_layout_passes=False),
        scratch_types=[
            pltpu.VMEM((_CHUNK,), jnp.int32),
            pltpu.VMEM((_CHUNK,), jnp.int32),
            pltpu.VMEM((_CHUNK,), jnp.float32),
            pltpu.VMEM((_CHUNK, _D), jnp.float32),
            pltpu.VMEM((_ZC, _D), jnp.float32),
            pltpu.VMEM_SHARED((_NP, _D), jnp.float32),
            pltpu.SemaphoreType.DMA,
        ],
    )


def _dense_body(p0_ref, p1_ref, wt_ref, b_ref, s_ref, h_out, s_out):
    agg = p0_ref[...] + p1_ref[...]
    hn = jnp.dot(agg, wt_ref[...], preferred_element_type=jnp.float32)
    hn = jnp.maximum(hn + b_ref[...], 0.0)
    h_out[...] = hn
    s_out[...] = s_ref[...] + hn


_BLK = 1000


def _tc_dense(p0, p1, w_t, b, s):
    return pl.pallas_call(
        _dense_body,
        grid=(_N // _BLK,),
        in_specs=[
            pl.BlockSpec((_BLK, _D), lambda i: (i, 0)),
            pl.BlockSpec((_BLK, _D), lambda i: (i, 0)),
            pl.BlockSpec((_D, _D), lambda i: (0, 0)),
            pl.BlockSpec((1, _D), lambda i: (0, 0)),
            pl.BlockSpec((_BLK, _D), lambda i: (i, 0)),
        ],
        out_specs=[
            pl.BlockSpec((_BLK, _D), lambda i: (i, 0)),
            pl.BlockSpec((_BLK, _D), lambda i: (i, 0)),
        ],
        out_shape=[
            jax.ShapeDtypeStruct((_N, _D), jnp.float32),
            jax.ShapeDtypeStruct((_N, _D), jnp.float32),
        ],
    )(p0, p1, w_t, b, s)


def kernel(h, edge_index, edge_weight, W0, b0, W1, b1, W2, b2):
    dst = edge_index[0].reshape(_NW, _NCHUNK, _CHUNK)
    src = edge_index[1].reshape(_NW, _NCHUNK, _CHUNK)
    w = edge_weight.reshape(_NW, _NCHUNK, _CHUNK)

    sum_h = jnp.zeros((_N, _D), jnp.float32)
    for (W, b) in ((W0, b0), (W1, b1), (W2, b2)):
        parts = _sc_spmm()(h, src, dst, w)
        p0 = parts[0, :_N]
        p1 = parts[1, :_N]
        h, sum_h = _tc_dense(p0, p1, W.T,
                             b.reshape(1, _D), sum_h)
    return sum_h


# speedup vs baseline: 6.2332x; 1.9430x over previous
import functools

import jax
import jax.numpy as jnp
from jax import lax
from jax.experimental import pallas as pl
from jax.experimental.pallas import tpu as pltpu
from jax.experimental.pallas import tpu_sc as plsc

_N = 10000
_E = 320000
_D = 128

_NC = 2
_NS = 16
_NW = _NC * _NS
_EPW = _E // _NW
_CHUNK = 80
_NCHUNK = _EPW // _CHUNK
_NP = 10240
_RPS = _NP // _NS
_ZC = 64
_NZ = _RPS // _ZC
_WBC = 128
_NWB = _RPS // _WBC
_UNROLL = 4


def _spmm_body(h_hbm, pk_hbm, w_hbm, out_hbm,
               pk0, pk1, wc0, wc1, rows0, rows1, stage_v, acc_sh,
               se0, se1, sg0, sg1):
    cid = lax.axis_index("c")
    sid = lax.axis_index("s")
    wid = sid * _NC + cid

    def stage(k, pk_b, wc_b, sem):
        pltpu.async_copy(pk_hbm.at[wid, k], pk_b, sem)
        pltpu.async_copy(w_hbm.at[wid, k], wc_b, sem)

    def wait_stage(pk_b, wc_b, sem):
        pltpu.make_async_copy(pk_hbm.at[wid, 0], pk_b, sem).wait()
        pltpu.make_async_copy(w_hbm.at[wid, 0], wc_b, sem).wait()

    def gather(pk_b, rows_b, sem):
        pltpu.async_copy(h_hbm.at[pk_b.at[0]], rows_b, sem)

    def wait_gather(pk_b, rows_b, sem):
        pltpu.make_async_copy(h_hbm.at[pk_b.at[0]], rows_b, sem).wait()

    def process(pk_b, wc_b, rows_b):
        def rowgrp(q, _):
            for e in range(_UNROLL):
                i = q * _UNROLL + e
                wv = plsc.load_gather(
                    wc_b, [jnp.full((16,), i, jnp.int32)])
                for j in range(_D // 16):
                    sl = pl.ds(j * 16, 16)
                    rows_b[i, sl] = rows_b[i, sl] * wv
            return 0
        lax.fori_loop(0, _CHUNK // _UNROLL, rowgrp, 0)
        pltpu.sync_copy(rows_b, acc_sh.at[pk_b.at[1]], add=True)

    stage(0, pk0, wc0, se0)
    stage(1, pk1, wc1, se1)

    def zrow(i, _):
        z = jnp.zeros((16,), jnp.float32)
        for j in range(_D // 16):
            stage_v[i, pl.ds(j * 16, 16)] = z
        return 0
    lax.fori_loop(0, _ZC, zrow, 0)
    for k in range(_NZ):
        pltpu.sync_copy(stage_v, acc_sh.at[pl.ds(sid * _RPS + k * _ZC, _ZC)])
    plsc.subcore_barrier()

    wait_stage(pk0, wc0, se0)
    gather(pk0, rows0, sg0)

    def step(g, pk_a, wc_a, rows_a, se_a, sg_a, pk_b, wc_b, rows_b,
             se_b, sg_b):
        wait_gather(pk_a, rows_a, sg_a)

        @pl.when(g + 1 < _NCHUNK)
        def _():
            wait_stage(pk_b, wc_b, se_b)
            gather(pk_b, rows_b, sg_b)

        process(pk_a, wc_a, rows_a)

        @pl.when(g + 2 < _NCHUNK)
        def _():
            stage(g + 2, pk_a, wc_a, se_a)

    def body(g, _):
        @pl.when(g % 2 == 0)
        def _():
            step(g, pk0, wc0, rows0, se0, sg0, pk1, wc1, rows1, se1, sg1)

        @pl.when(g % 2 == 1)
        def _():
            step(g, pk1, wc1, rows1, se1, sg1, pk0, wc0, rows0, se0, sg0)
        return 0
    lax.fori_loop(0, _NCHUNK, body, 0)
    plsc.subcore_barrier()

    for k in range(_NWB):
        sl = pl.ds(sid * _RPS + k * _WBC, _WBC)
        pltpu.sync_copy(acc_sh.at[sl], out_hbm.at[cid, sl])


@functools.cache
def _sc_spmm():
    return pl.kernel(
        _spmm_body,
        out_type=jax.ShapeDtypeStruct((_NC, _NP, _D), jnp.float32),
        mesh=plsc.VectorSubcoreMesh(core_axis_name="c",
                                    subcore_axis_name="s"),
        compiler_params=pltpu.CompilerParams(needs_layout_passes=False),
        scratch_types=[
            pltpu.VMEM((2, _CHUNK), jnp.int32),
            pltpu.VMEM((2, _CHUNK), jnp.int32),
            pltpu.VMEM((_CHUNK,), jnp.float32),
            pltpu.VMEM((_CHUNK,), jnp.float32),
            pltpu.VMEM((_CHUNK, _D), jnp.float32),
            pltpu.VMEM((_CHUNK, _D), jnp.float32),
            pltpu.VMEM((_ZC, _D), jnp.float32),
            pltpu.VMEM_SHARED((_NP, _D), jnp.float32),
            pltpu.SemaphoreType.DMA,
            pltpu.SemaphoreType.DMA,
            pltpu.SemaphoreType.DMA,
            pltpu.SemaphoreType.DMA,
        ],
    )


def _dense_body(p0_ref, p1_ref, wt_ref, b_ref, s_ref, h_out, s_out):
    agg = p0_ref[...] + p1_ref[...]
    hn = jnp.dot(agg, wt_ref[...], preferred_element_type=jnp.float32)
    hn = jnp.maximum(hn + b_ref[...], 0.0)
    h_out[...] = hn
    s_out[...] = s_ref[...] + hn


_BLK = 1000


def _tc_dense(p0, p1, w_t, b, s):
    return pl.pallas_call(
        _dense_body,
        grid=(_N // _BLK,),
        in_specs=[
            pl.BlockSpec((_BLK, _D), lambda i: (i, 0)),
            pl.BlockSpec((_BLK, _D), lambda i: (i, 0)),
            pl.BlockSpec((_D, _D), lambda i: (0, 0)),
            pl.BlockSpec((1, _D), lambda i: (0, 0)),
            pl.BlockSpec((_BLK, _D), lambda i: (i, 0)),
        ],
        out_specs=[
            pl.BlockSpec((_BLK, _D), lambda i: (i, 0)),
            pl.BlockSpec((_BLK, _D), lambda i: (i, 0)),
        ],
        out_shape=[
            jax.ShapeDtypeStruct((_N, _D), jnp.float32),
            jax.ShapeDtypeStruct((_N, _D), jnp.float32),
        ],
    )(p0, p1, w_t, b, s)


def kernel(h, edge_index, edge_weight, W0, b0, W1, b1, W2, b2):
    dst = edge_index[0].reshape(_NW, _NCHUNK, _CHUNK)
    src = edge_index[1].reshape(_NW, _NCHUNK, _CHUNK)
    pk = jnp.stack([src, dst], axis=2)
    w = edge_weight.reshape(_NW, _NCHUNK, _CHUNK)

    sum_h = jnp.zeros((_N, _D), jnp.float32)
    for (W, b) in ((W0, b0), (W1, b1), (W2, b2)):
        parts = _sc_spmm()(h, pk, w)
        p0 = parts[0, :_N]
        p1 = parts[1, :_N]
        h, sum_h = _tc_dense(p0, p1, W.T,
                             b.reshape(1, _D), sum_h)
    return sum_h


# speedup vs baseline: 7.7748x; 1.2473x over previous
import functools

import jax
import jax.numpy as jnp
from jax import lax
from jax.experimental import pallas as pl
from jax.experimental.pallas import tpu as pltpu
from jax.experimental.pallas import tpu_sc as plsc

_N = 10000
_E = 320000
_D = 128

_NC = 2
_NS = 16
_NW = _NC * _NS
_EPW = _E // _NW
_CHUNK = 80
_NCHUNK = _EPW // _CHUNK
_NP = 10240
_RPS = _NP // _NS
_ZC = 80
_NZ = _RPS // _ZC
_WBC = 128
_NWB = _RPS // _WBC
_UNROLL = 4
_NSLOT = 3


def _spmm_body(h_hbm, pk_hbm, w_hbm, out_hbm, *scratch):
    pk = scratch[0:3]
    wc = scratch[3:6]
    rows = scratch[6:9]
    acc_sh = scratch[9]
    se = scratch[10:13]
    sg = scratch[13:16]
    ss = scratch[16:19]

    cid = lax.axis_index("c")
    sid = lax.axis_index("s")
    wid = sid * _NC + cid

    def stage(k, s):
        pltpu.async_copy(pk_hbm.at[wid, k], pk[s], se[s])
        pltpu.async_copy(w_hbm.at[wid, k], wc[s], se[s])

    def wait_stage(s):
        pltpu.make_async_copy(pk_hbm.at[wid, 0], pk[s], se[s]).wait()
        pltpu.make_async_copy(w_hbm.at[wid, 0], wc[s], se[s]).wait()

    def gather(s):
        pltpu.async_copy(h_hbm.at[pk[s].at[0]], rows[s], sg[s])

    def wait_gather(s):
        pltpu.make_async_copy(h_hbm.at[pk[s].at[0]], rows[s], sg[s]).wait()

    def scatter(s):
        pltpu.async_copy(rows[s], acc_sh.at[pk[s].at[1]], ss[s], add=True)

    def wait_scatter(s):
        pltpu.make_async_copy(rows[s], acc_sh.at[pk[s].at[1]],
                              ss[s]).wait()

    def scale(s):
        rows_b = rows[s]
        wc_b = wc[s]

        def rowgrp(q, _):
            for e in range(_UNROLL):
                i = q * _UNROLL + e
                wv = plsc.load_gather(
                    wc_b, [jnp.full((16,), i, jnp.int32)])
                for j in range(_D // 16):
                    sl = pl.ds(j * 16, 16)
                    rows_b[i, sl] = rows_b[i, sl] * wv
            return 0
        lax.fori_loop(0, _CHUNK // _UNROLL, rowgrp, 0)

    stage(0, 0)
    stage(1, 1)

    def zrow(i, _):
        z = jnp.zeros((16,), jnp.float32)
        for j in range(_D // 16):
            rows[0][i, pl.ds(j * 16, 16)] = z
        return 0
    lax.fori_loop(0, _ZC, zrow, 0)
    for k in range(_NZ):
        pltpu.sync_copy(rows[0],
                        acc_sh.at[pl.ds(sid * _RPS + k * _ZC, _ZC)])
    plsc.subcore_barrier()

    wait_stage(0)
    gather(0)

    def step(g, a, b, c):
        wait_gather(a)

        @pl.when(g + 1 < _NCHUNK)
        def _():
            wait_stage(b)
            gather(b)

        scale(a)
        scatter(a)

        @pl.when(g >= 1)
        def _():
            wait_scatter(c)

        @pl.when(g + 2 < _NCHUNK)
        def _():
            stage(g + 2, c)

    def body(g, _):
        for r in range(_NSLOT):
            @pl.when(g % _NSLOT == r)
            def _(r=r):
                step(g, r, (r + 1) % _NSLOT, (r + 2) % _NSLOT)
        return 0
    lax.fori_loop(0, _NCHUNK, body, 0)
    wait_scatter((_NCHUNK - 1) % _NSLOT)
    plsc.subcore_barrier()

    for k in range(_NWB):
        sl = pl.ds(sid * _RPS + k * _WBC, _WBC)
        pltpu.sync_copy(acc_sh.at[sl], out_hbm.at[cid, sl])


@functools.cache
def _sc_spmm():
    return pl.kernel(
        _spmm_body,
        out_type=jax.ShapeDtypeStruct((_NC, _NP, _D), jnp.float32),
        mesh=plsc.VectorSubcoreMesh(core_axis_name="c",
                                    subcore_axis_name="s"),
        compiler_params=pltpu.CompilerParams(needs_layout_passes=False),
        scratch_types=(
            [pltpu.VMEM((2, _CHUNK), jnp.int32) for _ in range(_NSLOT)]
            + [pltpu.VMEM((_CHUNK,), jnp.float32) for _ in range(_NSLOT)]
            + [pltpu.VMEM((_CHUNK, _D), jnp.float32) for _ in range(_NSLOT)]
            + [pltpu.VMEM_SHARED((_NP, _D), jnp.float32)]
            + [pltpu.SemaphoreType.DMA for _ in range(3 * _NSLOT)]
        ),
    )


def _dense_body(p0_ref, p1_ref, wt_ref, b_ref, s_ref, h_out, s_out):
    agg = p0_ref[...] + p1_ref[...]
    hn = jnp.dot(agg, wt_ref[...], preferred_element_type=jnp.float32)
    hn = jnp.maximum(hn + b_ref[...], 0.0)
    h_out[...] = hn
    s_out[...] = s_ref[...] + hn


_BLK = 1000


def _tc_dense(p0, p1, w_t, b, s):
    return pl.pallas_call(
        _dense_body,
        grid=(_N // _BLK,),
        in_specs=[
            pl.BlockSpec((_BLK, _D), lambda i: (i, 0)),
            pl.BlockSpec((_BLK, _D), lambda i: (i, 0)),
            pl.BlockSpec((_D, _D), lambda i: (0, 0)),
            pl.BlockSpec((1, _D), lambda i: (0, 0)),
            pl.BlockSpec((_BLK, _D), lambda i: (i, 0)),
        ],
        out_specs=[
            pl.BlockSpec((_BLK, _D), lambda i: (i, 0)),
            pl.BlockSpec((_BLK, _D), lambda i: (i, 0)),
        ],
        out_shape=[
            jax.ShapeDtypeStruct((_N, _D), jnp.float32),
            jax.ShapeDtypeStruct((_N, _D), jnp.float32),
        ],
    )(p0, p1, w_t, b, s)


def kernel(h, edge_index, edge_weight, W0, b0, W1, b1, W2, b2):
    dst = edge_index[0].reshape(_NW, _NCHUNK, _CHUNK)
    src = edge_index[1].reshape(_NW, _NCHUNK, _CHUNK)
    pk = jnp.stack([src, dst], axis=2)
    w = edge_weight.reshape(_NW, _NCHUNK, _CHUNK)

    sum_h = jnp.zeros((_N, _D), jnp.float32)
    for (W, b) in ((W0, b0), (W1, b1), (W2, b2)):
        parts = _sc_spmm()(h, pk, w)
        p0 = parts[0, :_N]
        p1 = parts[1, :_N]
        h, sum_h = _tc_dense(p0, p1, W.T,
                             b.reshape(1, _D), sum_h)
    return sum_h


# speedup vs baseline: 7.8034x; 1.0037x over previous
import functools

import jax
import jax.numpy as jnp
from jax import lax
from jax.experimental import pallas as pl
from jax.experimental.pallas import tpu as pltpu
from jax.experimental.pallas import tpu_sc as plsc

_N = 10000
_E = 320000
_D = 128

_NC = 2
_NS = 16
_NW = _NC * _NS
_EPW = _E // _NW
_CHUNK = 80
_NCHUNK = _EPW // _CHUNK
_NP = 10240
_RPS = _NP // _NS
_ZC = 80
_NZ = _RPS // _ZC
_WBC = 128
_NWB = _RPS // _WBC
_UNROLL = 4
_NSLOT = 4


def _spmm_body(h_hbm, pk_hbm, w_hbm, out_hbm, *scratch):
    pk = scratch[0:4]
    wc = scratch[4:8]
    rows = scratch[8:12]
    acc_sh = scratch[12]
    se = scratch[13:17]
    sg = scratch[17:21]
    ss = scratch[21:25]

    cid = lax.axis_index("c")
    sid = lax.axis_index("s")
    wid = sid * _NC + cid

    def stage(k, s):
        pltpu.async_copy(pk_hbm.at[wid, k], pk[s], se[s])
        pltpu.async_copy(w_hbm.at[wid, k], wc[s], se[s])

    def wait_stage(s):
        pltpu.make_async_copy(pk_hbm.at[wid, 0], pk[s], se[s]).wait()
        pltpu.make_async_copy(w_hbm.at[wid, 0], wc[s], se[s]).wait()

    def gather(s):
        pltpu.async_copy(h_hbm.at[pk[s].at[0]], rows[s], sg[s])

    def wait_gather(s):
        pltpu.make_async_copy(h_hbm.at[pk[s].at[0]], rows[s], sg[s]).wait()

    def scatter(s):
        pltpu.async_copy(rows[s], acc_sh.at[pk[s].at[1]], ss[s], add=True)

    def wait_scatter(s):
        pltpu.make_async_copy(rows[s], acc_sh.at[pk[s].at[1]],
                              ss[s]).wait()

    def scale(s):
        rows_b = rows[s]
        wc_b = wc[s]

        def rowgrp(q, _):
            for e in range(_UNROLL):
                i = q * _UNROLL + e
                wv = plsc.load_gather(
                    wc_b, [jnp.full((16,), i, jnp.int32)])
                for j in range(_D // 16):
                    sl = pl.ds(j * 16, 16)
                    rows_b[i, sl] = rows_b[i, sl] * wv
            return 0
        lax.fori_loop(0, _CHUNK // _UNROLL, rowgrp, 0)

    stage(0, 0)
    stage(1, 1)
    stage(2, 2)

    def zrow(i, _):
        z = jnp.zeros((16,), jnp.float32)
        for j in range(_D // 16):
            rows[0][i, pl.ds(j * 16, 16)] = z
        return 0
    lax.fori_loop(0, _ZC, zrow, 0)
    for k in range(_NZ):
        pltpu.sync_copy(rows[0],
                        acc_sh.at[pl.ds(sid * _RPS + k * _ZC, _ZC)])
    plsc.subcore_barrier()

    wait_stage(0)
    gather(0)
    wait_stage(1)
    gather(1)

    def step(g, a, c, d):
        wait_gather(a)

        @pl.when(g + 2 < _NCHUNK)
        def _():
            wait_stage(c)
            gather(c)

        scale(a)
        scatter(a)

        @pl.when(g >= 1)
        def _():
            wait_scatter(d)

        @pl.when(g + 3 < _NCHUNK)
        def _():
            stage(g + 3, d)

    def body(g, _):
        for r in range(_NSLOT):
            @pl.when(g % _NSLOT == r)
            def _(r=r):
                step(g, r, (r + 2) % _NSLOT, (r + 3) % _NSLOT)
        return 0
    lax.fori_loop(0, _NCHUNK, body, 0)
    wait_scatter((_NCHUNK - 1) % _NSLOT)
    plsc.subcore_barrier()

    for k in range(_NWB):
        sl = pl.ds(sid * _RPS + k * _WBC, _WBC)
        pltpu.sync_copy(acc_sh.at[sl], out_hbm.at[cid, sl])


@functools.cache
def _sc_spmm():
    return pl.kernel(
        _spmm_body,
        out_type=jax.ShapeDtypeStruct((_NC, _NP, _D), jnp.float32),
        mesh=plsc.VectorSubcoreMesh(core_axis_name="c",
                                    subcore_axis_name="s"),
        compiler_params=pltpu.CompilerParams(needs_layout_passes=False),
        scratch_types=(
            [pltpu.VMEM((2, _CHUNK), jnp.int32) for _ in range(_NSLOT)]
            + [pltpu.VMEM((_CHUNK,), jnp.float32) for _ in range(_NSLOT)]
            + [pltpu.VMEM((_CHUNK, _D), jnp.float32) for _ in range(_NSLOT)]
            + [pltpu.VMEM_SHARED((_NP, _D), jnp.float32)]
            + [pltpu.SemaphoreType.DMA for _ in range(3 * _NSLOT)]
        ),
    )


def _dense_body(p0_ref, p1_ref, wt_ref, b_ref, s_ref, h_out, s_out):
    agg = p0_ref[...] + p1_ref[...]
    hn = jnp.dot(agg, wt_ref[...], preferred_element_type=jnp.float32)
    hn = jnp.maximum(hn + b_ref[...], 0.0)
    h_out[...] = hn
    s_out[...] = s_ref[...] + hn


_BLK = 1000


def _tc_dense(p0, p1, w_t, b, s):
    return pl.pallas_call(
        _dense_body,
        grid=(_N // _BLK,),
        in_specs=[
            pl.BlockSpec((_BLK, _D), lambda i: (i, 0)),
            pl.BlockSpec((_BLK, _D), lambda i: (i, 0)),
            pl.BlockSpec((_D, _D), lambda i: (0, 0)),
            pl.BlockSpec((1, _D), lambda i: (0, 0)),
            pl.BlockSpec((_BLK, _D), lambda i: (i, 0)),
        ],
        out_specs=[
            pl.BlockSpec((_BLK, _D), lambda i: (i, 0)),
            pl.BlockSpec((_BLK, _D), lambda i: (i, 0)),
        ],
        out_shape=[
            jax.ShapeDtypeStruct((_N, _D), jnp.float32),
            jax.ShapeDtypeStruct((_N, _D), jnp.float32),
        ],
    )(p0, p1, w_t, b, s)


def kernel(h, edge_index, edge_weight, W0, b0, W1, b1, W2, b2):
    dst = edge_index[0].reshape(_NW, _NCHUNK, _CHUNK)
    src = edge_index[1].reshape(_NW, _NCHUNK, _CHUNK)
    pk = jnp.stack([src, dst], axis=2)
    w = edge_weight.reshape(_NW, _NCHUNK, _CHUNK)

    sum_h = jnp.zeros((_N, _D), jnp.float32)
    for (W, b) in ((W0, b0), (W1, b1), (W2, b2)):
        parts = _sc_spmm()(h, pk, w)
        p0 = parts[0, :_N]
        p1 = parts[1, :_N]
        h, sum_h = _tc_dense(p0, p1, W.T,
                             b.reshape(1, _D), sum_h)
    return sum_h


# speedup vs baseline: 8.1811x; 1.0484x over previous
import functools

import jax
import jax.numpy as jnp
from jax import lax
from jax.experimental import pallas as pl
from jax.experimental.pallas import tpu as pltpu
from jax.experimental.pallas import tpu_sc as plsc

_N = 10000
_E = 320000
_D = 128

_NC = 2
_NS = 16
_NW = _NC * _NS
_EPW = _E // _NW
_CHUNK = 80
_NCHUNK = _EPW // _CHUNK
_NP = 10240
_RPS = _NP // _NS
_ZC = 80
_NZ = _RPS // _ZC
_WBC = 128
_NWB = _RPS // _WBC
_UNROLL = 4
_NSLOT = 4


def _spmm_body(h_hbm, pk_hbm, w_hbm, out_hbm, *scratch):
    pk = scratch[0:4]
    wc = scratch[4:8]
    rows = scratch[8:12]
    acc_sh = scratch[12]
    se = scratch[13:17]
    sg = scratch[17:21]
    ss = scratch[21:25]

    cid = lax.axis_index("c")
    sid = lax.axis_index("s")
    wid = sid * _NC + cid

    def stage(k, s):
        pltpu.async_copy(pk_hbm.at[wid, k], pk[s], se[s])
        pltpu.async_copy(w_hbm.at[wid, k], wc[s], se[s])

    def wait_stage(s):
        pltpu.make_async_copy(pk_hbm.at[wid, 0], pk[s], se[s]).wait()
        pltpu.make_async_copy(w_hbm.at[wid, 0], wc[s], se[s]).wait()

    def gather(s):
        pltpu.async_copy(h_hbm.at[pk[s].at[0]], rows[s], sg[s])

    def wait_gather(s):
        pltpu.make_async_copy(h_hbm.at[pk[s].at[0]], rows[s], sg[s]).wait()

    def scatter(s):
        pltpu.async_copy(rows[s], acc_sh.at[pk[s].at[1]], ss[s], add=True)

    def wait_scatter(s):
        pltpu.make_async_copy(rows[s], acc_sh.at[pk[s].at[1]],
                              ss[s]).wait()

    def scale(s):
        rows_b = rows[s]
        wc_b = wc[s]

        def rowgrp(q, _):
            for e in range(_UNROLL):
                i = q * _UNROLL + e
                wv = plsc.load_gather(
                    wc_b, [jnp.full((16,), i, jnp.int32)])
                for j in range(_D // 16):
                    sl = pl.ds(j * 16, 16)
                    rows_b[i, sl] = rows_b[i, sl] * wv
            return 0
        lax.fori_loop(0, _CHUNK // _UNROLL, rowgrp, 0)

    stage(0, 0)
    stage(1, 1)
    stage(2, 2)

    def zrow(i, _):
        z = jnp.zeros((16,), jnp.float32)
        for j in range(_D // 16):
            rows[0][i, pl.ds(j * 16, 16)] = z
        return 0
    lax.fori_loop(0, _ZC, zrow, 0)
    for k in range(_NZ):
        pltpu.sync_copy(rows[0],
                        acc_sh.at[pl.ds(sid * _RPS + k * _ZC, _ZC)])
    plsc.subcore_barrier()

    wait_stage(0)
    gather(0)
    wait_stage(1)
    gather(1)

    def step(g, a, c, d):
        wait_gather(a)

        @pl.when(g + 2 < _NCHUNK)
        def _():
            wait_stage(c)
            gather(c)

        scale(a)
        scatter(a)

        @pl.when(g >= 1)
        def _():
            wait_scatter(d)

        @pl.when(g + 3 < _NCHUNK)
        def _():
            stage(g + 3, d)

    def body(g, _):
        for r in range(_NSLOT):
            @pl.when(g % _NSLOT == r)
            def _(r=r):
                step(g, r, (r + 2) % _NSLOT, (r + 3) % _NSLOT)
        return 0
    lax.fori_loop(0, _NCHUNK, body, 0)
    wait_scatter((_NCHUNK - 1) % _NSLOT)
    plsc.subcore_barrier()

    for k in range(_NWB):
        sl = pl.ds(sid * _RPS + k * _WBC, _WBC)
        pltpu.sync_copy(acc_sh.at[sl], out_hbm.at[cid, sl])


@functools.cache
def _sc_spmm():
    return pl.kernel(
        _spmm_body,
        out_type=jax.ShapeDtypeStruct((_NC, _NP, _D), jnp.float32),
        mesh=plsc.VectorSubcoreMesh(core_axis_name="c",
                                    subcore_axis_name="s"),
        compiler_params=pltpu.CompilerParams(needs_layout_passes=False),
        scratch_types=(
            [pltpu.VMEM((2, _CHUNK), jnp.int32) for _ in range(_NSLOT)]
            + [pltpu.VMEM((_CHUNK,), jnp.float32) for _ in range(_NSLOT)]
            + [pltpu.VMEM((_CHUNK, _D), jnp.float32) for _ in range(_NSLOT)]
            + [pltpu.VMEM_SHARED((_NP, _D), jnp.float32)]
            + [pltpu.SemaphoreType.DMA for _ in range(3 * _NSLOT)]
        ),
    )


def _dense_body(p_ref, wt_ref, b_ref, s_ref, h_out, s_out):
    agg = p_ref[0] + p_ref[1]
    hn = jnp.dot(agg, wt_ref[...], preferred_element_type=jnp.float32)
    hn = jnp.maximum(hn + b_ref[...], 0.0)
    h_out[...] = hn
    s_out[...] = s_ref[...] + hn


_BLK = 2000


def _tc_dense(parts, w_t, b, s):
    return pl.pallas_call(
        _dense_body,
        grid=(_N // _BLK,),
        in_specs=[
            pl.BlockSpec((_NC, _BLK, _D), lambda i: (0, i, 0)),
            pl.BlockSpec((_D, _D), lambda i: (0, 0)),
            pl.BlockSpec((1, _D), lambda i: (0, 0)),
            pl.BlockSpec((_BLK, _D), lambda i: (i, 0)),
        ],
        out_specs=[
            pl.BlockSpec((_BLK, _D), lambda i: (i, 0)),
            pl.BlockSpec((_BLK, _D), lambda i: (i, 0)),
        ],
        out_shape=[
            jax.ShapeDtypeStruct((_N, _D), jnp.float32),
            jax.ShapeDtypeStruct((_N, _D), jnp.float32),
        ],
    )(parts, w_t, b, s)


def kernel(h, edge_index, edge_weight, W0, b0, W1, b1, W2, b2):
    dst = edge_index[0].reshape(_NW, _NCHUNK, _CHUNK)
    src = edge_index[1].reshape(_NW, _NCHUNK, _CHUNK)
    pk = jnp.stack([src, dst], axis=2)
    w = edge_weight.reshape(_NW, _NCHUNK, _CHUNK)

    sum_h = jnp.zeros((_N, _D), jnp.float32)
    for (W, b) in ((W0, b0), (W1, b1), (W2, b2)):
        parts = _sc_spmm()(h, pk, w)
        h, sum_h = _tc_dense(parts, W.T, b.reshape(1, _D), sum_h)
    return sum_h
